# baseline (speedup 1.0000x reference)
import functools

import jax
import jax.numpy as jnp
from jax import lax
from jax.experimental import pallas as pl
from jax.experimental.pallas import tpu as pltpu
from jax.experimental.pallas import tpu_sc as plsc

_N = 10000
_D = 128
_E = 320000
_NW = 32
_EPW = _E // _NW
_B = 40
_NCH = _EPW // _B
_NPAD = 10240
_RPT = _NPAD // 16


def _fill(ref, rows, value):
    def _row(i, carry):
        def _col(j, c2):
            ref[i, pl.ds(j * 16, 16)] = jnp.full((16,), value, jnp.float32)
            return c2
        return lax.fori_loop(0, _D // 16, _col, carry)
    lax.fori_loop(0, rows, _row, 0)


def _make_sc_body(with_deg):
    def body(*refs):
        if with_deg:
            (hq, hp, s0, d0, s1, d1, s2, d2, s3, d3, out, dout,
             acc, src1d, dst1d, rb0, rb1, rb2, rb3, rb4,
             g0, g1, g2, g3, g4, x0, x1, x2, x3, x4) = refs
        else:
            (hq, hp, s0, d0, s1, d1, s2, d2, s3, d3, out,
             acc, src1d, dst1d, rb0, rb1, rb2, rb3, rb4,
             g0, g1, g2, g3, g4, x0, x1, x2, x3, x4) = refs
        cid = lax.axis_index("c")
        tid = lax.axis_index("s")
        w = cid * 16 + tid
        rows4 = (rb0, rb1, rb2, rb3, rb4)
        gsem = (g0, g1, g2, g3, g4)
        ssem = (x0, x1, x2, x3, x4)

        def _sca(k, sem, src_buf):
            return pltpu.async_copy(
                src_buf, acc.at[dst1d.at[pl.ds(k * _B, _B)]], sem, add=True)

        def _sca_wait(sem, src_buf):
            pltpu.make_async_copy(
                src_buf, acc.at[dst1d.at[pl.ds(0, _B)]], sem).wait()

        tables = (hq, hp, hp, hq)
        edges = ((s0, d0), (s1, d1), (s2, d2), (s3, d3))
        passes = [(e, False) for e in range(4)]
        if with_deg:
            passes += [(e, True) for e in range(4)]
        for e, is_deg in passes:
            _fill(rb0, _B, 0.0)
            for z in range(_RPT // _B):
                pltpu.async_copy(
                    rb0, acc.at[pl.ds(tid * _RPT + z * _B, _B)], gsem[z % 5])
            for z in range(_RPT // _B):
                pltpu.make_async_copy(
                    rb0, acc.at[pl.ds(tid * _RPT + (z % 5) * _B, _B)],
                    gsem[z % 5]).wait()
            plsc.subcore_barrier()
            se, de = edges[e]
            tab = tables[e]
            pltpu.sync_copy(de.at[w], dst1d)

            if is_deg:
                _fill(rb0, _B, 1.0)
                def _dgrp(k0, carry):
                    for b in range(5):
                        k = k0 * 5 + b

                        @pl.when((k >= 5) & (k - 5 < _NCH))
                        def _():
                            _sca_wait(ssem[b], rb0)

                        @pl.when(k < _NCH)
                        def _():
                            _sca(k, ssem[b], rb0)
                    return carry
                lax.fori_loop(0, (_NCH + 5 + 4) // 5, _dgrp, 0)
            else:
                pltpu.sync_copy(se.at[w], src1d)
                def _ring(k0, carry):
                    for b in range(5):
                        k = k0 * 5 + b
                        bb = (b + 1) % 5

                        @pl.when((k >= 5) & (k - 5 < _NCH))
                        def _():
                            _sca_wait(ssem[b], rows4[b])

                        @pl.when(k < _NCH)
                        def _():
                            pltpu.async_copy(
                                tab.at[src1d.at[pl.ds(k * _B, _B)]],
                                rows4[b], gsem[b])

                        @pl.when((k >= 4) & (k - 4 < _NCH))
                        def _():
                            pltpu.make_async_copy(
                                tab.at[src1d.at[pl.ds((k - 4) * _B, _B)]],
                                rows4[bb], gsem[bb]).wait()
                            _sca(k - 4, ssem[bb], rows4[bb])
                    return carry
                lax.fori_loop(0, (_NCH + 5 + 4) // 5, _ring, 0)
            plsc.subcore_barrier()
            tgt = dout if is_deg else out
            roff = tid * _RPT
            pltpu.sync_copy(acc.at[pl.ds(roff, _RPT)],
                            tgt.at[cid, e, pl.ds(roff, _RPT)])
            plsc.subcore_barrier()
    return body


_agg_shape = jax.ShapeDtypeStruct((2, 4, _NPAD, _D), jnp.float32)
_sc_scratch = [
    pltpu.VMEM_SHARED((_NPAD, _D), jnp.float32),
    pltpu.VMEM((_EPW,), jnp.int32),
    pltpu.VMEM((_EPW,), jnp.int32),
    pltpu.VMEM((_B, _D), jnp.float32),
    pltpu.VMEM((_B, _D), jnp.float32),
    pltpu.VMEM((_B, _D), jnp.float32),
    pltpu.VMEM((_B, _D), jnp.float32),
    pltpu.VMEM((_B, _D), jnp.float32),
] + [pltpu.SemaphoreType.DMA] * 10
_mesh = plsc.VectorSubcoreMesh(core_axis_name="c", subcore_axis_name="s")

_sc_agg_deg = pl.kernel(
    _make_sc_body(True),
    out_type=[_agg_shape, _agg_shape],
    mesh=_mesh,
    scratch_types=_sc_scratch,
)

_sc_agg = pl.kernel(
    _make_sc_body(False),
    out_type=[_agg_shape],
    mesh=_mesh,
    scratch_types=_sc_scratch,
)

_R = 1000


def _tc_body(relu, hq_ref, hp_ref, pp_ref, dd_ref, wsq, wsp, wn0, wn1, wn2,
             wn3, bq, bp, oq_ref, op_ref):
    def mean_of(e):
        a = pp_ref[0, e] + pp_ref[1, e]
        deg = dd_ref[0, e][:, 0:1] + dd_ref[1, e][:, 0:1]
        return a / jnp.maximum(deg, 1.0)

    m0 = mean_of(0)
    m1 = mean_of(1)
    m2 = mean_of(2)
    m3 = mean_of(3)
    f32 = jnp.float32
    nq = (jnp.dot(hq_ref[...], wsq[...], preferred_element_type=f32)
          + jnp.dot(m1, wn1[...], preferred_element_type=f32)
          + jnp.dot(m2, wn2[...], preferred_element_type=f32) + bq[...])
    np_ = (jnp.dot(hp_ref[...], wsp[...], preferred_element_type=f32)
           + jnp.dot(m0, wn0[...], preferred_element_type=f32)
           + jnp.dot(m3, wn3[...], preferred_element_type=f32) + bp[...])
    if relu:
        nq = jnp.maximum(nq, 0.0)
        np_ = jnp.maximum(np_, 0.0)
    oq_ref[...] = nq
    op_ref[...] = np_


def _tc_layer(relu, hq, hp, pp, dd, wsq, wsp, wn0, wn1, wn2, wn3, bq, bp):
    blk = pl.BlockSpec((_R, _D), lambda i: (i, 0))
    pspec = pl.BlockSpec((2, 4, _R, _D), lambda i: (0, 0, i, 0))
    wspec = pl.BlockSpec((_D, _D), lambda i: (0, 0))
    bspec = pl.BlockSpec((1, _D), lambda i: (0, 0))
    return pl.pallas_call(
        functools.partial(_tc_body, relu),
        grid=(_N // _R,),
        in_specs=[blk, blk, pspec, pspec,
                  wspec, wspec, wspec, wspec, wspec, wspec, bspec, bspec],
        out_specs=[blk, blk],
        out_shape=[jax.ShapeDtypeStruct((_N, _D), jnp.float32)] * 2,
    )(hq, hp, pp, dd, wsq, wsp, wn0, wn1, wn2, wn3, bq, bp)


def kernel(x_query, x_product, edge_click, edge_rclick, edge_rqr, edge_qr,
           W_self, W_neigh, bias):
    def _split(edge):
        return (edge[0].astype(jnp.int32).reshape(_NW, _EPW),
                edge[1].astype(jnp.int32).reshape(_NW, _EPW))

    s0, d0 = _split(edge_click)
    s1, d1 = _split(edge_rclick)
    s2, d2 = _split(edge_rqr)
    s3, d3 = _split(edge_qr)

    hq, hp = x_query, x_product
    dd = None
    for l in range(2):
        if l == 0:
            pp, dd = _sc_agg_deg(hq, hp, s0, d0, s1, d1, s2, d2, s3, d3)
        else:
            (pp,) = _sc_agg(hq, hp, s0, d0, s1, d1, s2, d2, s3, d3)
        wsq = W_self[l, 1] + W_self[l, 2]
        wsp = W_self[l, 0] + W_self[l, 3]
        bq = (bias[l, 1] + bias[l, 2]).reshape(1, _D)
        bp = (bias[l, 0] + bias[l, 3]).reshape(1, _D)
        hq, hp = _tc_layer(l == 0, hq, hp, pp, dd,
                           wsq, wsp,
                           W_neigh[l, 0], W_neigh[l, 1], W_neigh[l, 2],
                           W_neigh[l, 3], bq, bp)
    return hq, hp


# speedup vs baseline: 1.0123x; 1.0123x over previous
import functools

import jax
import jax.numpy as jnp
from jax import lax
from jax.experimental import pallas as pl
from jax.experimental.pallas import tpu as pltpu
from jax.experimental.pallas import tpu_sc as plsc

_N = 10000
_D = 128
_E = 320000
_NW = 32
_EPW = _E // _NW
_B = 40
_NCH = _EPW // _B
_NPAD = 10240
_RPT = _NPAD // 16


def _fill(ref, rows, value):
    def _row(i, carry):
        def _col(j, c2):
            ref[i, pl.ds(j * 16, 16)] = jnp.full((16,), value, jnp.float32)
            return c2
        return lax.fori_loop(0, _D // 16, _col, carry)
    lax.fori_loop(0, rows, _row, 0)


def _make_sc_body(with_deg):
    def body(*refs):
        if with_deg:
            (hq, hp, s0, d0, s1, d1, s2, d2, s3, d3, out, dout,
             acc, src1d, dst1d, rb0, rb1, rb2, rb3, rb4,
             g0, g1, g2, g3, g4, x0, x1, x2, x3, x4) = refs
        else:
            (hq, hp, s0, d0, s1, d1, s2, d2, s3, d3, out,
             acc, src1d, dst1d, rb0, rb1, rb2, rb3, rb4,
             g0, g1, g2, g3, g4, x0, x1, x2, x3, x4) = refs
        cid = lax.axis_index("c")
        tid = lax.axis_index("s")
        w = cid * 16 + tid
        rows4 = (rb0, rb1, rb2, rb3, rb4)
        gsem = (g0, g1, g2, g3, g4)
        ssem = (x0, x1, x2, x3, x4)

        def _sca(k, sem, src_buf):
            return pltpu.async_copy(
                src_buf, acc.at[dst1d.at[pl.ds(k * _B, _B)]], sem, add=True)

        def _sca_wait(sem, src_buf):
            pltpu.make_async_copy(
                src_buf, acc.at[dst1d.at[pl.ds(0, _B)]], sem).wait()

        tables = (hq, hp, hp, hq)
        edges = ((s0, d0), (s1, d1), (s2, d2), (s3, d3))
        passes = [(e, False) for e in range(4)]
        if with_deg:
            passes += [(e, True) for e in range(4)]
        roff = tid * _RPT
        for pi, (e, is_deg) in enumerate(passes):
            if pi > 0:
                pe, pdeg = passes[pi - 1]
                ptgt = dout if pdeg else out
                pltpu.sync_copy(acc.at[pl.ds(roff, _RPT)],
                                ptgt.at[cid, pe, pl.ds(roff, _RPT)])
            _fill(rb0, _B, 0.0)
            for z in range(_RPT // _B):
                pltpu.async_copy(
                    rb0, acc.at[pl.ds(tid * _RPT + z * _B, _B)], gsem[z % 5])
            se, de = edges[e]
            tab = tables[e]
            pltpu.sync_copy(de.at[w], dst1d)
            for z in range(_RPT // _B):
                pltpu.make_async_copy(
                    rb0, acc.at[pl.ds(tid * _RPT + (z % 5) * _B, _B)],
                    gsem[z % 5]).wait()
            plsc.subcore_barrier()

            if is_deg:
                _fill(rb0, _B, 1.0)
                def _dgrp(k0, carry):
                    for b in range(5):
                        k = k0 * 5 + b

                        @pl.when((k >= 5) & (k - 5 < _NCH))
                        def _():
                            _sca_wait(ssem[b], rb0)

                        @pl.when(k < _NCH)
                        def _():
                            _sca(k, ssem[b], rb0)
                    return carry
                lax.fori_loop(0, (_NCH + 5 + 4) // 5, _dgrp, 0)
            else:
                pltpu.sync_copy(se.at[w], src1d)
                def _ring(k0, carry):
                    for b in range(5):
                        k = k0 * 5 + b
                        bb = (b + 1) % 5

                        @pl.when((k >= 5) & (k - 5 < _NCH))
                        def _():
                            _sca_wait(ssem[b], rows4[b])

                        @pl.when(k < _NCH)
                        def _():
                            pltpu.async_copy(
                                tab.at[src1d.at[pl.ds(k * _B, _B)]],
                                rows4[b], gsem[b])

                        @pl.when((k >= 4) & (k - 4 < _NCH))
                        def _():
                            pltpu.make_async_copy(
                                tab.at[src1d.at[pl.ds((k - 4) * _B, _B)]],
                                rows4[bb], gsem[bb]).wait()
                            _sca(k - 4, ssem[bb], rows4[bb])
                    return carry
                lax.fori_loop(0, (_NCH + 5 + 4) // 5, _ring, 0)
            plsc.subcore_barrier()
        fe, fdeg = passes[-1]
        ftgt = dout if fdeg else out
        pltpu.sync_copy(acc.at[pl.ds(roff, _RPT)],
                        ftgt.at[cid, fe, pl.ds(roff, _RPT)])
    return body


_agg_shape = jax.ShapeDtypeStruct((2, 4, _NPAD, _D), jnp.float32)
_sc_scratch = [
    pltpu.VMEM_SHARED((_NPAD, _D), jnp.float32),
    pltpu.VMEM((_EPW,), jnp.int32),
    pltpu.VMEM((_EPW,), jnp.int32),
    pltpu.VMEM((_B, _D), jnp.float32),
    pltpu.VMEM((_B, _D), jnp.float32),
    pltpu.VMEM((_B, _D), jnp.float32),
    pltpu.VMEM((_B, _D), jnp.float32),
    pltpu.VMEM((_B, _D), jnp.float32),
] + [pltpu.SemaphoreType.DMA] * 10
_mesh = plsc.VectorSubcoreMesh(core_axis_name="c", subcore_axis_name="s")

_sc_agg_deg = pl.kernel(
    _make_sc_body(True),
    out_type=[_agg_shape, _agg_shape],
    mesh=_mesh,
    scratch_types=_sc_scratch,
)

_sc_agg = pl.kernel(
    _make_sc_body(False),
    out_type=[_agg_shape],
    mesh=_mesh,
    scratch_types=_sc_scratch,
)

_R = 1000


def _tc_body(relu, hq_ref, hp_ref, pp_ref, dd_ref, wsq, wsp, wn0, wn1, wn2,
             wn3, bq, bp, oq_ref, op_ref):
    def mean_of(e):
        a = pp_ref[0, e] + pp_ref[1, e]
        deg = dd_ref[0, e][:, 0:1] + dd_ref[1, e][:, 0:1]
        return a / jnp.maximum(deg, 1.0)

    m0 = mean_of(0)
    m1 = mean_of(1)
    m2 = mean_of(2)
    m3 = mean_of(3)
    f32 = jnp.float32
    nq = (jnp.dot(hq_ref[...], wsq[...], preferred_element_type=f32)
          + jnp.dot(m1, wn1[...], preferred_element_type=f32)
          + jnp.dot(m2, wn2[...], preferred_element_type=f32) + bq[...])
    np_ = (jnp.dot(hp_ref[...], wsp[...], preferred_element_type=f32)
           + jnp.dot(m0, wn0[...], preferred_element_type=f32)
           + jnp.dot(m3, wn3[...], preferred_element_type=f32) + bp[...])
    if relu:
        nq = jnp.maximum(nq, 0.0)
        np_ = jnp.maximum(np_, 0.0)
    oq_ref[...] = nq
    op_ref[...] = np_


def _tc_layer(relu, hq, hp, pp, dd, wsq, wsp, wn0, wn1, wn2, wn3, bq, bp):
    blk = pl.BlockSpec((_R, _D), lambda i: (i, 0))
    pspec = pl.BlockSpec((2, 4, _R, _D), lambda i: (0, 0, i, 0))
    wspec = pl.BlockSpec((_D, _D), lambda i: (0, 0))
    bspec = pl.BlockSpec((1, _D), lambda i: (0, 0))
    return pl.pallas_call(
        functools.partial(_tc_body, relu),
        grid=(_N // _R,),
        in_specs=[blk, blk, pspec, pspec,
                  wspec, wspec, wspec, wspec, wspec, wspec, bspec, bspec],
        out_specs=[blk, blk],
        out_shape=[jax.ShapeDtypeStruct((_N, _D), jnp.float32)] * 2,
    )(hq, hp, pp, dd, wsq, wsp, wn0, wn1, wn2, wn3, bq, bp)


def kernel(x_query, x_product, edge_click, edge_rclick, edge_rqr, edge_qr,
           W_self, W_neigh, bias):
    def _split(edge):
        return (edge[0].astype(jnp.int32).reshape(_NW, _EPW),
                edge[1].astype(jnp.int32).reshape(_NW, _EPW))

    s0, d0 = _split(edge_click)
    s1, d1 = _split(edge_rclick)
    s2, d2 = _split(edge_rqr)
    s3, d3 = _split(edge_qr)

    hq, hp = x_query, x_product
    dd = None
    for l in range(2):
        if l == 0:
            pp, dd = _sc_agg_deg(hq, hp, s0, d0, s1, d1, s2, d2, s3, d3)
        else:
            (pp,) = _sc_agg(hq, hp, s0, d0, s1, d1, s2, d2, s3, d3)
        wsq = W_self[l, 1] + W_self[l, 2]
        wsp = W_self[l, 0] + W_self[l, 3]
        bq = (bias[l, 1] + bias[l, 2]).reshape(1, _D)
        bp = (bias[l, 0] + bias[l, 3]).reshape(1, _D)
        hq, hp = _tc_layer(l == 0, hq, hp, pp, dd,
                           wsq, wsp,
                           W_neigh[l, 0], W_neigh[l, 1], W_neigh[l, 2],
                           W_neigh[l, 3], bq, bp)
    return hq, hp
